# baseline (speedup 1.0000x reference)
import jax
import jax.numpy as jnp
from jax import lax
from jax.experimental import pallas as pl
from jax.experimental.pallas import tpu as pltpu
from jax.experimental.pallas import tpu_sc as plsc

_LANES = 16
_CH = 128
_NBUF = 5
_NC = 2
_NS = 16


def kernel(lyrics_ids, mask, table):
    B, S = lyrics_ids.shape
    V, D = table.shape
    N = B * S
    n_units = _NC * _NS
    per_w = N // n_units
    n_ch = per_w // _CH
    n_outer = n_ch // _NBUF

    ids = lyrics_ids.reshape(N).astype(jnp.int32)
    msk = mask.reshape(N).astype(jnp.int32)

    mesh = plsc.VectorSubcoreMesh(core_axis_name="c", subcore_axis_name="s")

    @pl.kernel(
        out_type=jax.ShapeDtypeStruct((N, D), table.dtype),
        mesh=mesh,
        scratch_types=[
            pltpu.VMEM((per_w,), jnp.int32),
            pltpu.VMEM((per_w,), jnp.int32),
            pltpu.VMEM((_NBUF, _CH, D), jnp.float32),
            pltpu.SemaphoreType.DMA((_NBUF,)),
            pltpu.SemaphoreType.DMA((_NBUF,)),
        ],
    )
    def sc_gather(table_hbm, ids_hbm, msk_hbm, out_hbm, idx_v, msk_v, rows_v,
                  gsem, wsem):
        wid = lax.axis_index("c") * _NS + lax.axis_index("s")
        base = wid * per_w

        pltpu.sync_copy(ids_hbm.at[pl.ds(base, per_w)], idx_v)
        pltpu.sync_copy(msk_hbm.at[pl.ds(base, per_w)], msk_v)

        @pl.loop(0, per_w, step=_LANES)
        def _(c):
            slc = pl.ds(c, _LANES)
            idx_v.at[slc][...] = idx_v.at[slc][...] * msk_v.at[slc][...]

        def gather(chunk, b):
            return pltpu.make_async_copy(
                table_hbm.at[idx_v.at[pl.ds(chunk * _CH, _CH)]],
                rows_v.at[b],
                gsem.at[b],
            )

        def writeback(chunk, b):
            return pltpu.make_async_copy(
                rows_v.at[b],
                out_hbm.at[pl.ds(base + chunk * _CH, _CH)],
                wsem.at[b],
            )

        for b in range(_NBUF):
            gather(b, b).start()

        @pl.loop(0, n_outer)
        def _(o):
            for b in range(_NBUF):
                chunk = o * _NBUF + b
                gather(chunk, b).wait()
                writeback(chunk, b).start()
            for b in range(_NBUF):
                chunk = o * _NBUF + b
                writeback(chunk, b).wait()

                @pl.when(o < n_outer - 1)
                def _():
                    gather(chunk + _NBUF, b).start()

    out = sc_gather(table, ids, msk)
    return out.reshape(B, S, D)


# speedup vs baseline: 27.9731x; 27.9731x over previous
import jax
import jax.numpy as jnp
from jax import lax
from jax.experimental import pallas as pl
from jax.experimental.pallas import tpu as pltpu
from jax.experimental.pallas import tpu_sc as plsc

_LANES = 16
_CH = 128
_NBUF = 5
_NC = 2
_NS = 16
_PAD = 1024


def kernel(lyrics_ids, mask, table):
    B, S = lyrics_ids.shape
    V, D = table.shape
    N = B * S
    n_units = _NC * _NS
    per_w = N // n_units
    n_ch = per_w // _CH
    n_outer = n_ch // _NBUF

    ids = lyrics_ids.reshape(N).astype(jnp.int32)
    msk = mask.reshape(N).astype(jnp.int32)
    pad = (V + jnp.arange(N, dtype=jnp.int32) % _PAD).astype(jnp.int32)
    table_x = jnp.concatenate(
        [table, jnp.broadcast_to(table[0], (_PAD, D))], axis=0
    )

    mesh = plsc.VectorSubcoreMesh(core_axis_name="c", subcore_axis_name="s")

    @pl.kernel(
        out_type=jax.ShapeDtypeStruct((N, D), table.dtype),
        mesh=mesh,
        scratch_types=[
            pltpu.VMEM((per_w,), jnp.int32),
            pltpu.VMEM((per_w,), jnp.int32),
            pltpu.VMEM((per_w,), jnp.int32),
            pltpu.VMEM((_NBUF, _CH, D), jnp.float32),
            pltpu.SemaphoreType.DMA((_NBUF,)),
            pltpu.SemaphoreType.DMA((_NBUF,)),
        ],
    )
    def sc_gather(table_hbm, ids_hbm, msk_hbm, pad_hbm, out_hbm,
                  idx_v, msk_v, pad_v, rows_v, gsem, wsem):
        wid = lax.axis_index("c") * _NS + lax.axis_index("s")
        base = wid * per_w

        pltpu.sync_copy(ids_hbm.at[pl.ds(base, per_w)], idx_v)
        pltpu.sync_copy(msk_hbm.at[pl.ds(base, per_w)], msk_v)
        pltpu.sync_copy(pad_hbm.at[pl.ds(base, per_w)], pad_v)

        @pl.loop(0, per_w, step=_LANES)
        def _(c):
            slc = pl.ds(c, _LANES)
            p = pad_v.at[slc][...]
            idx_v.at[slc][...] = p + (idx_v.at[slc][...] - p) * msk_v.at[slc][...]

        def gather(chunk, b):
            return pltpu.make_async_copy(
                table_hbm.at[idx_v.at[pl.ds(chunk * _CH, _CH)]],
                rows_v.at[b],
                gsem.at[b],
            )

        def writeback(chunk, b):
            return pltpu.make_async_copy(
                rows_v.at[b],
                out_hbm.at[pl.ds(base + chunk * _CH, _CH)],
                wsem.at[b],
            )

        for b in range(_NBUF):
            gather(b, b).start()

        @pl.loop(0, n_outer)
        def _(o):
            for b in range(_NBUF):
                chunk = o * _NBUF + b
                gather(chunk, b).wait()
                writeback(chunk, b).start()
            for b in range(_NBUF):
                chunk = o * _NBUF + b
                writeback(chunk, b).wait()

                @pl.when(o < n_outer - 1)
                def _():
                    gather(chunk + _NBUF, b).start()

    out = sc_gather(table_x, ids, msk, pad)
    return out.reshape(B, S, D)


# speedup vs baseline: 29.0238x; 1.0376x over previous
import jax
import jax.numpy as jnp
from jax import lax
from jax.experimental import pallas as pl
from jax.experimental.pallas import tpu as pltpu
from jax.experimental.pallas import tpu_sc as plsc

_LANES = 16
_CH = 128
_NBUF = 5
_NC = 2
_NS = 16
_PAD = 1024


def kernel(lyrics_ids, mask, table):
    B, S = lyrics_ids.shape
    V, D = table.shape
    N = B * S
    n_units = _NC * _NS
    per_w = N // n_units
    n_ch = per_w // _CH
    n_outer = n_ch // _NBUF

    ids = lyrics_ids.reshape(N).astype(jnp.int32)
    msk = mask.reshape(N).astype(jnp.int32)
    table_x = jnp.concatenate(
        [table, jnp.broadcast_to(table[0], (_PAD, D))], axis=0
    )

    mesh = plsc.VectorSubcoreMesh(core_axis_name="c", subcore_axis_name="s")

    @pl.kernel(
        out_type=jax.ShapeDtypeStruct((N, D), table.dtype),
        mesh=mesh,
        scratch_types=[
            pltpu.VMEM((per_w,), jnp.int32),
            pltpu.VMEM((per_w,), jnp.int32),
            pltpu.VMEM((_NBUF, _CH, D), jnp.float32),
            pltpu.SemaphoreType.DMA((_NBUF,)),
            pltpu.SemaphoreType.DMA((_NBUF,)),
        ],
    )
    def sc_gather(table_hbm, ids_hbm, msk_hbm, out_hbm,
                  idx_v, msk_v, rows_v, gsem, wsem):
        wid = lax.axis_index("c") * _NS + lax.axis_index("s")
        base = wid * per_w
        iota = lax.iota(jnp.int32, _LANES)

        pltpu.sync_copy(ids_hbm.at[pl.ds(base, per_w)], idx_v)
        pltpu.sync_copy(msk_hbm.at[pl.ds(base, per_w)], msk_v)

        def mask_chunk(chunk):
            for g in range(_CH // _LANES):
                c = chunk * _CH + g * _LANES
                slc = pl.ds(c, _LANES)
                p = V + ((c + wid * 32) & (_PAD - 1)) + iota
                idx_v.at[slc][...] = (
                    p + (idx_v.at[slc][...] - p) * msk_v.at[slc][...]
                )

        def gather(chunk, b):
            return pltpu.make_async_copy(
                table_hbm.at[idx_v.at[pl.ds(chunk * _CH, _CH)]],
                rows_v.at[b],
                gsem.at[b],
            )

        def writeback(chunk, b):
            return pltpu.make_async_copy(
                rows_v.at[b],
                out_hbm.at[pl.ds(base + chunk * _CH, _CH)],
                wsem.at[b],
            )

        for b in range(_NBUF):
            mask_chunk(b)
            gather(b, b).start()

        @pl.loop(0, n_outer)
        def _(o):
            for b in range(_NBUF):
                chunk = o * _NBUF + b
                gather(chunk, b).wait()
                writeback(chunk, b).start()
            for b in range(_NBUF):
                chunk = o * _NBUF + b
                writeback(chunk, b).wait()

                @pl.when(o < n_outer - 1)
                def _():
                    mask_chunk(chunk + _NBUF)
                    gather(chunk + _NBUF, b).start()

    out = sc_gather(table_x, ids, msk)
    return out.reshape(B, S, D)


# speedup vs baseline: 29.1826x; 1.0055x over previous
import jax
import jax.numpy as jnp
from jax import lax
from jax.experimental import pallas as pl
from jax.experimental.pallas import tpu as pltpu
from jax.experimental.pallas import tpu_sc as plsc

_LANES = 16
_CH = 64
_NBUF = 10
_NC = 2
_NS = 16
_PAD = 1024


def kernel(lyrics_ids, mask, table):
    B, S = lyrics_ids.shape
    V, D = table.shape
    N = B * S
    n_units = _NC * _NS
    per_w = N // n_units
    n_ch = per_w // _CH
    n_outer = n_ch // _NBUF

    ids = lyrics_ids.reshape(N).astype(jnp.int32)
    msk = mask.reshape(N).astype(jnp.int32)
    table_x = jnp.concatenate(
        [table, jnp.broadcast_to(table[0], (_PAD, D))], axis=0
    )

    mesh = plsc.VectorSubcoreMesh(core_axis_name="c", subcore_axis_name="s")

    @pl.kernel(
        out_type=jax.ShapeDtypeStruct((N, D), table.dtype),
        mesh=mesh,
        scratch_types=[
            pltpu.VMEM((per_w,), jnp.int32),
            pltpu.VMEM((per_w,), jnp.int32),
            pltpu.VMEM((_NBUF, _CH, D), jnp.float32),
            pltpu.SemaphoreType.DMA((_NBUF,)),
            pltpu.SemaphoreType.DMA((_NBUF,)),
        ],
    )
    def sc_gather(table_hbm, ids_hbm, msk_hbm, out_hbm,
                  idx_v, msk_v, rows_v, gsem, wsem):
        wid = lax.axis_index("c") * _NS + lax.axis_index("s")
        base = wid * per_w
        iota = lax.iota(jnp.int32, _LANES)

        pltpu.sync_copy(ids_hbm.at[pl.ds(base, per_w)], idx_v)
        pltpu.sync_copy(msk_hbm.at[pl.ds(base, per_w)], msk_v)

        def mask_chunk(chunk):
            for g in range(_CH // _LANES):
                c = chunk * _CH + g * _LANES
                slc = pl.ds(c, _LANES)
                p = V + ((c + wid * 32) & (_PAD - 1)) + iota
                idx_v.at[slc][...] = (
                    p + (idx_v.at[slc][...] - p) * msk_v.at[slc][...]
                )

        def gather(chunk, b):
            return pltpu.make_async_copy(
                table_hbm.at[idx_v.at[pl.ds(chunk * _CH, _CH)]],
                rows_v.at[b],
                gsem.at[b],
            )

        def writeback(chunk, b):
            return pltpu.make_async_copy(
                rows_v.at[b],
                out_hbm.at[pl.ds(base + chunk * _CH, _CH)],
                wsem.at[b],
            )

        for b in range(_NBUF):
            mask_chunk(b)
            gather(b, b).start()

        @pl.loop(0, n_outer)
        def _(o):
            for b in range(_NBUF):
                chunk = o * _NBUF + b
                gather(chunk, b).wait()
                writeback(chunk, b).start()
            for b in range(_NBUF):
                chunk = o * _NBUF + b
                writeback(chunk, b).wait()

                @pl.when(o < n_outer - 1)
                def _():
                    mask_chunk(chunk + _NBUF)
                    gather(chunk + _NBUF, b).start()

    out = sc_gather(table_x, ids, msk)
    return out.reshape(B, S, D)


# speedup vs baseline: 30.1055x; 1.0316x over previous
import jax
import jax.numpy as jnp
from jax import lax
from jax.experimental import pallas as pl
from jax.experimental.pallas import tpu as pltpu
from jax.experimental.pallas import tpu_sc as plsc

_LANES = 16
_CH = 64
_NBUF = 10
_NC = 2
_NS = 16
_PAD = 8192


def kernel(lyrics_ids, mask, table):
    B, S = lyrics_ids.shape
    V, D = table.shape
    N = B * S
    n_units = _NC * _NS
    per_w = N // n_units
    n_ch = per_w // _CH
    n_outer = n_ch // _NBUF

    ids = lyrics_ids.reshape(N).astype(jnp.int32)
    msk = mask.reshape(N).astype(jnp.int32)
    table_x = jnp.concatenate(
        [table, jnp.broadcast_to(table[0], (_PAD, D))], axis=0
    )

    mesh = plsc.VectorSubcoreMesh(core_axis_name="c", subcore_axis_name="s")

    @pl.kernel(
        out_type=jax.ShapeDtypeStruct((N, D), table.dtype),
        mesh=mesh,
        scratch_types=[
            pltpu.VMEM((per_w,), jnp.int32),
            pltpu.VMEM((per_w,), jnp.int32),
            pltpu.VMEM((_NBUF, _CH, D), jnp.float32),
            pltpu.SemaphoreType.DMA((_NBUF,)),
            pltpu.SemaphoreType.DMA((_NBUF,)),
        ],
    )
    def sc_gather(table_hbm, ids_hbm, msk_hbm, out_hbm,
                  idx_v, msk_v, rows_v, gsem, wsem):
        wid = lax.axis_index("c") * _NS + lax.axis_index("s")
        base = wid * per_w
        iota = lax.iota(jnp.int32, _LANES)

        pltpu.sync_copy(ids_hbm.at[pl.ds(base, per_w)], idx_v)
        pltpu.sync_copy(msk_hbm.at[pl.ds(base, per_w)], msk_v)

        def mask_chunk(chunk):
            for g in range(_CH // _LANES):
                c = chunk * _CH + g * _LANES
                slc = pl.ds(c, _LANES)
                p = V + ((c + wid * 32) & (_PAD - 1)) + iota
                idx_v.at[slc][...] = (
                    p + (idx_v.at[slc][...] - p) * msk_v.at[slc][...]
                )

        def gather(chunk, b):
            return pltpu.make_async_copy(
                table_hbm.at[idx_v.at[pl.ds(chunk * _CH, _CH)]],
                rows_v.at[b],
                gsem.at[b],
            )

        def writeback(chunk, b):
            return pltpu.make_async_copy(
                rows_v.at[b],
                out_hbm.at[pl.ds(base + chunk * _CH, _CH)],
                wsem.at[b],
            )

        for b in range(_NBUF):
            mask_chunk(b)
            gather(b, b).start()

        @pl.loop(0, n_outer)
        def _(o):
            for b in range(_NBUF):
                chunk = o * _NBUF + b
                gather(chunk, b).wait()
                writeback(chunk, b).start()
            for b in range(_NBUF):
                chunk = o * _NBUF + b
                writeback(chunk, b).wait()

                @pl.when(o < n_outer - 1)
                def _():
                    mask_chunk(chunk + _NBUF)
                    gather(chunk + _NBUF, b).start()

    out = sc_gather(table_x, ids, msk)
    return out.reshape(B, S, D)


# speedup vs baseline: 30.1143x; 1.0003x over previous
import jax
import jax.numpy as jnp
from jax import lax
from jax.experimental import pallas as pl
from jax.experimental.pallas import tpu as pltpu
from jax.experimental.pallas import tpu_sc as plsc

_LANES = 16
_CH = 80
_NBUF = 10
_NC = 2
_NS = 16
_PAD = 8192


def kernel(lyrics_ids, mask, table):
    B, S = lyrics_ids.shape
    V, D = table.shape
    N = B * S
    n_units = _NC * _NS
    per_w = N // n_units
    n_ch = per_w // _CH
    n_outer = n_ch // _NBUF

    ids = lyrics_ids.reshape(N).astype(jnp.int32)
    msk = mask.reshape(N).astype(jnp.int32)
    table_x = jnp.concatenate(
        [table, jnp.broadcast_to(table[0], (_PAD, D))], axis=0
    )

    mesh = plsc.VectorSubcoreMesh(core_axis_name="c", subcore_axis_name="s")

    @pl.kernel(
        out_type=jax.ShapeDtypeStruct((N, D), table.dtype),
        mesh=mesh,
        scratch_types=[
            pltpu.VMEM((per_w,), jnp.int32),
            pltpu.VMEM((per_w,), jnp.int32),
            pltpu.VMEM((_NBUF, _CH, D), jnp.float32),
            pltpu.SemaphoreType.DMA((_NBUF,)),
            pltpu.SemaphoreType.DMA((_NBUF,)),
        ],
    )
    def sc_gather(table_hbm, ids_hbm, msk_hbm, out_hbm,
                  idx_v, msk_v, rows_v, gsem, wsem):
        wid = lax.axis_index("c") * _NS + lax.axis_index("s")
        base = wid * per_w
        iota = lax.iota(jnp.int32, _LANES)

        pltpu.sync_copy(ids_hbm.at[pl.ds(base, per_w)], idx_v)
        pltpu.sync_copy(msk_hbm.at[pl.ds(base, per_w)], msk_v)

        def mask_chunk(chunk):
            for g in range(_CH // _LANES):
                c = chunk * _CH + g * _LANES
                slc = pl.ds(c, _LANES)
                p = V + ((c + wid * 32) & (_PAD - 1)) + iota
                idx_v.at[slc][...] = (
                    p + (idx_v.at[slc][...] - p) * msk_v.at[slc][...]
                )

        def gather(chunk, b):
            return pltpu.make_async_copy(
                table_hbm.at[idx_v.at[pl.ds(chunk * _CH, _CH)]],
                rows_v.at[b],
                gsem.at[b],
            )

        def writeback(chunk, b):
            return pltpu.make_async_copy(
                rows_v.at[b],
                out_hbm.at[pl.ds(base + chunk * _CH, _CH)],
                wsem.at[b],
            )

        for b in range(_NBUF):
            mask_chunk(b)
            gather(b, b).start()

        @pl.loop(0, n_outer)
        def _(o):
            for b in range(_NBUF):
                chunk = o * _NBUF + b
                gather(chunk, b).wait()
                writeback(chunk, b).start()
            for b in range(_NBUF):
                chunk = o * _NBUF + b
                writeback(chunk, b).wait()

                @pl.when(o < n_outer - 1)
                def _():
                    mask_chunk(chunk + _NBUF)
                    gather(chunk + _NBUF, b).start()

    out = sc_gather(table_x, ids, msk)
    return out.reshape(B, S, D)


# speedup vs baseline: 30.2936x; 1.0060x over previous
import jax
import jax.numpy as jnp
from jax import lax
from jax.experimental import pallas as pl
from jax.experimental.pallas import tpu as pltpu
from jax.experimental.pallas import tpu_sc as plsc

_LANES = 16
_CH = 80
_NBUF = 10
_NC = 2
_NS = 16
_PAD = 8192


def kernel(lyrics_ids, mask, table):
    B, S = lyrics_ids.shape
    V, D = table.shape
    N = B * S
    n_units = _NC * _NS
    per_w = N // n_units
    n_ch = per_w // _CH
    n_outer = n_ch // _NBUF

    ids = lyrics_ids.reshape(N).astype(jnp.int32)
    msk = mask.reshape(N).astype(jnp.int32)
    table_x = jnp.concatenate(
        [table, jnp.broadcast_to(table[0], (_PAD, D))], axis=0
    )

    mesh = plsc.VectorSubcoreMesh(core_axis_name="c", subcore_axis_name="s")

    @pl.kernel(
        out_type=jax.ShapeDtypeStruct((N, D), table.dtype),
        mesh=mesh,
        scratch_types=[
            pltpu.VMEM((per_w,), jnp.int32),
            pltpu.VMEM((per_w,), jnp.int32),
            pltpu.VMEM((_NBUF, _CH, D), jnp.float32),
            pltpu.SemaphoreType.DMA((_NBUF,)),
            pltpu.SemaphoreType.DMA((_NBUF,)),
        ],
    )
    def sc_gather(table_hbm, ids_hbm, msk_hbm, out_hbm,
                  idx_v, msk_v, rows_v, gsem, wsem):
        wid = lax.axis_index("c") * _NS + lax.axis_index("s")
        base = wid * per_w
        iota = lax.iota(jnp.int32, _LANES)

        ld_ids = pltpu.make_async_copy(
            ids_hbm.at[pl.ds(base, per_w)], idx_v, gsem.at[0]
        )
        ld_msk = pltpu.make_async_copy(
            msk_hbm.at[pl.ds(base, per_w)], msk_v, gsem.at[1]
        )
        ld_ids.start()
        ld_msk.start()
        ld_ids.wait()
        ld_msk.wait()

        def mask_chunk(chunk):
            for g in range(_CH // _LANES):
                c = chunk * _CH + g * _LANES
                slc = pl.ds(c, _LANES)
                p = V + ((c + wid * 32) & (_PAD - 1)) + iota
                idx_v.at[slc][...] = (
                    p + (idx_v.at[slc][...] - p) * msk_v.at[slc][...]
                )

        def gather(chunk, b):
            return pltpu.make_async_copy(
                table_hbm.at[idx_v.at[pl.ds(chunk * _CH, _CH)]],
                rows_v.at[b],
                gsem.at[b],
            )

        def writeback(chunk, b):
            return pltpu.make_async_copy(
                rows_v.at[b],
                out_hbm.at[pl.ds(base + chunk * _CH, _CH)],
                wsem.at[b],
            )

        for b in range(_NBUF):
            mask_chunk(b)
            gather(b, b).start()

        @pl.loop(0, n_outer)
        def _(o):
            for b in range(_NBUF):
                chunk = o * _NBUF + b
                gather(chunk, b).wait()
                writeback(chunk, b).start()
            for b in range(_NBUF):
                chunk = o * _NBUF + b
                writeback(chunk, b).wait()

                @pl.when(o < n_outer - 1)
                def _():
                    mask_chunk(chunk + _NBUF)
                    gather(chunk + _NBUF, b).start()

    out = sc_gather(table_x, ids, msk)
    return out.reshape(B, S, D)
